# baseline (speedup 1.0000x reference)
import functools

import jax
import jax.numpy as jnp
from jax import lax
from jax.experimental import pallas as pl
from jax.experimental.pallas import tpu as pltpu
from jax.experimental.pallas import tpu_sc as plsc

_D = 64
_NROWS = 4096 * 200
_NW = 32
_ROWS_PER_W = _NROWS // _NW
_CHUNK = 400
_NCHUNK = _ROWS_PER_W // _CHUNK
_GROUPS = _CHUNK // 16

_mesh = plsc.VectorSubcoreMesh(core_axis_name="c", subcore_axis_name="s")


@functools.partial(
    pl.kernel,
    out_type=jax.ShapeDtypeStruct((_NROWS * _D,), jnp.float32),
    mesh=_mesh,
    scratch_types=[
        pltpu.VMEM((1000 * _D,), jnp.float32),
        pltpu.VMEM((51, _D), jnp.float32),
        pltpu.VMEM((20, _D), jnp.float32),
        pltpu.VMEM((2, _CHUNK), jnp.float32),
        pltpu.VMEM((2, _CHUNK * _D), jnp.float32),
        pltpu.SemaphoreType.DMA,
        pltpu.SemaphoreType.DMA,
    ],
    compiler_params=pltpu.CompilerParams(
        use_tc_tiling_on_sc=False, needs_layout_passes=False),
)
def _embed_sc(x_hbm, emb1_hbm, emb2_hbm, out_hbm, tab_v, e1_v, e2_v, x_v,
              rows_v, sem0, sem1):
    wid = lax.axis_index("s") * 2 + lax.axis_index("c")
    w_base = wid * _ROWS_PER_W
    sems = (sem0, sem1)

    pltpu.sync_copy(emb1_hbm, e1_v)
    pltpu.sync_copy(emb2_hbm, e2_v)

    def build_row(i, _):
        e1 = [e1_v[i, pl.ds(16 * k, 16)] for k in range(4)]
        base = i * (20 * _D)
        for j in range(20):
            for k in range(4):
                tab_v[pl.ds(base + j * _D + 16 * k, 16)] = (
                    e1[k] + e2_v[j, pl.ds(16 * k, 16)])
        return 0

    lax.fori_loop(0, 50, build_row, 0)

    lane = lax.iota(jnp.int32, 16)

    def do_chunk(g, slot):
        base = w_base + g * _CHUNK
        pltpu.sync_copy(x_hbm.at[pl.ds(base, _CHUNK)], x_v.at[slot])

        def group(t, _):
            xv = x_v[slot, pl.ds(t * 16, 16)]
            v = xv * 1000.0
            tr = v.astype(jnp.int32)
            frac = v - tr.astype(jnp.float32)
            up = (frac > 0.5) | ((frac == 0.5) & ((tr & 1) == 1))
            idx = jnp.minimum(tr + jnp.where(up, 1, 0), 999)
            addr_in = idx << 6
            row_addr = ((t * 16 + lane) << 6)
            vals0 = plsc.load_gather(tab_v, [addr_in])
            plsc.store_scatter(rows_v.at[slot], [row_addr], vals0)
            for j in range(1, _D):
                vals = plsc.load_gather(tab_v, [addr_in + j])
                plsc.store_scatter(rows_v.at[slot], [row_addr + j], vals)
            return 0

        lax.fori_loop(0, _GROUPS, group, 0)
        pltpu.make_async_copy(
            rows_v.at[slot], out_hbm.at[pl.ds(base * _D, _CHUNK * _D)],
            sems[slot],
        ).start()

    def outer(go, _):
        for b in range(2):
            g = go * 2 + b

            @pl.when(go > 0)
            def _wait_prev():
                pltpu.make_async_copy(
                    rows_v.at[b],
                    out_hbm.at[
                        pl.ds((w_base + (g - 2) * _CHUNK) * _D, _CHUNK * _D)],
                    sems[b],
                ).wait()

            do_chunk(g, b)
        return 0

    lax.fori_loop(0, _NCHUNK // 2, outer, 0)
    for b in range(2):
        pltpu.make_async_copy(
            rows_v.at[b],
            out_hbm.at[
                pl.ds((w_base + (_NCHUNK - 2 + b) * _CHUNK) * _D, _CHUNK * _D)],
            sems[b],
        ).wait()


def kernel(x, emb1, emb2):
    out = _embed_sc(x.reshape(-1), emb1, emb2)
    return out.reshape(4096, 200, _D)
